# baseline (speedup 1.0000x reference)
import functools
import math

import jax
import jax.numpy as jnp
from jax import lax
from jax.experimental import pallas as pl
from jax.experimental.pallas import tpu as pltpu
from jax.experimental.pallas import tpu_sc as plsc

_L = 16
_SUB = 8
_NBUF = 5


def _emb_kernel(n_seq, t_len, d):
    info = plsc.get_sparse_core_info()
    nc, ns = info.num_cores, info.num_subcores
    nw = nc * ns
    lanes = n_seq // nw
    assert n_seq == nw * lanes and lanes == 128 and d % _SUB == 0
    assert t_len % _NBUF == 0 and t_len // _NBUF >= 3
    scale = math.sqrt(d)

    mesh = plsc.VectorSubcoreMesh(core_axis_name="c", subcore_axis_name="s")

    @functools.partial(
        pl.kernel,
        out_type=jax.ShapeDtypeStruct(
            (t_len, d // _SUB, nw, _SUB, lanes), jnp.float32),
        mesh=mesh,
        compiler_params=pltpu.CompilerParams(
            use_tc_tiling_on_sc=False, needs_layout_passes=False,
            disable_bounds_checks=True),
        scratch_types=[
            pltpu.VMEM((t_len, lanes), jnp.int32),
            [pltpu.VMEM((lanes, d), jnp.float32) for _ in range(_NBUF)],
            [pltpu.VMEM((d // _SUB, _SUB, lanes), jnp.float32)
             for _ in range(_NBUF)],
            [pltpu.SemaphoreType.DMA for _ in range(_NBUF)],
            [pltpu.SemaphoreType.DMA for _ in range(_NBUF)],
        ],
    )
    def emb(xt_hbm, table_hbm, out_hbm, idx_v, gbuf, sbuf, gsem, ssem):
        wid = lax.axis_index("s") * nc + lax.axis_index("c")
        pltpu.sync_copy(xt_hbm.at[:, pl.ds(wid * lanes, lanes)], idx_v)

        def start_gather(t, b):
            pltpu.async_copy(table_hbm.at[idx_v.at[t]], gbuf[b], gsem[b])

        def wait_gather(b):
            pltpu.make_async_copy(
                table_hbm.at[idx_v.at[0]], gbuf[b], gsem[b]).wait()

        def start_scatter(t, b):
            pltpu.async_copy(sbuf[b], out_hbm.at[t, :, wid], ssem[b])

        def wait_scatter(b):
            pltpu.make_async_copy(
                sbuf[b], out_hbm.at[0, :, wid], ssem[b]).wait()

        dim_iota = lax.iota(jnp.int32, _L)
        zero_vec = jnp.zeros((_L,), jnp.int32)
        c_idx = [q * _L + dim_iota for q in range(d // _L)]

        def do_transpose(b):
            src, dst = gbuf[b], sbuf[b]

            @plsc.parallel_loop(0, lanes, unroll=8)
            def _(s):
                lane_vec = jnp.full((_L,), s, jnp.int32)
                row = src.at[s]
                for q in range(d // _L):
                    vals = row[pl.ds(q * _L, _L)] * scale
                    plsc.store_scatter(
                        dst, [zero_vec, c_idx[q], lane_vec], vals)

        for b in range(_NBUF):
            start_gather(b, b)
        for b in range(_NBUF):
            wait_gather(b)
            do_transpose(b)
            start_scatter(b, b)
            start_gather(b + _NBUF, b)
        @pl.loop(_NBUF, t_len - _NBUF, step=_NBUF)
        def _(t0):
            for b in range(_NBUF):
                t = t0 + b
                wait_scatter(b)
                wait_gather(b)
                do_transpose(b)
                start_scatter(t, b)
                start_gather(t + _NBUF, b)
        for b in range(_NBUF):
            t = t_len - _NBUF + b
            wait_scatter(b)
            wait_gather(b)
            do_transpose(b)
            start_scatter(t, b)
        for b in range(_NBUF):
            wait_scatter(b)

    return emb


def kernel(x, table):
    n_seq, t_len = x.shape
    d = table.shape[1]
    out5 = _emb_kernel(n_seq, t_len, d)(x.T.astype(jnp.int32), table)
    return jnp.transpose(out5, (2, 4, 0, 1, 3)).reshape(n_seq, t_len, d)


# speedup vs baseline: 1.7525x; 1.7525x over previous
import functools
import math

import jax
import jax.numpy as jnp
from jax import lax
from jax.experimental import pallas as pl
from jax.experimental.pallas import tpu as pltpu
from jax.experimental.pallas import tpu_sc as plsc

_L = 16
_SUB = 8
_NBUF = 5


def _emb_kernel(n_seq, t_len, d):
    info = plsc.get_sparse_core_info()
    nc, ns = info.num_cores, info.num_subcores
    nw = nc * ns
    lanes = n_seq // nw
    assert n_seq == nw * lanes and lanes == 128 and d % _SUB == 0
    assert t_len % _NBUF == 0 and t_len // _NBUF >= 3
    scale = math.sqrt(d)

    mesh = plsc.VectorSubcoreMesh(core_axis_name="c", subcore_axis_name="s")

    @functools.partial(
        pl.kernel,
        out_type=jax.ShapeDtypeStruct(
            (t_len, d // _SUB, nw, _SUB, lanes), jnp.float32),
        mesh=mesh,
        compiler_params=pltpu.CompilerParams(
            use_tc_tiling_on_sc=False, needs_layout_passes=False,
            disable_bounds_checks=True),
        scratch_types=[
            pltpu.VMEM((t_len, lanes), jnp.int32),
            [pltpu.VMEM((lanes, d), jnp.float32) for _ in range(_NBUF)],
            [pltpu.VMEM((d // _SUB, _SUB, lanes), jnp.float32)
             for _ in range(_NBUF)],
            [pltpu.SemaphoreType.DMA for _ in range(_NBUF)],
            [pltpu.SemaphoreType.DMA for _ in range(_NBUF)],
        ],
    )
    def emb(xt_hbm, table_hbm, out_hbm, idx_v, gbuf, sbuf, gsem, ssem):
        wid = lax.axis_index("s") * nc + lax.axis_index("c")
        pltpu.sync_copy(xt_hbm.at[:, pl.ds(wid * lanes, lanes)], idx_v)

        def start_gather(t, b):
            pltpu.async_copy(table_hbm.at[idx_v.at[t]], gbuf[b], gsem[b])

        def wait_gather(b):
            pltpu.make_async_copy(
                table_hbm.at[idx_v.at[0]], gbuf[b], gsem[b]).wait()

        def start_scatter(t, b):
            pltpu.async_copy(sbuf[b], out_hbm.at[t, :, wid], ssem[b])

        def wait_scatter(b):
            pltpu.make_async_copy(
                sbuf[b], out_hbm.at[0, :, wid], ssem[b]).wait()

        dim_iota = lax.iota(jnp.int32, _L)
        zero_vec = jnp.zeros((_L,), jnp.int32)
        cperm = [(dim_iota + k) % _L for k in range(_L)]
        wperm = [((dim_iota + k) % _L) * lanes + dim_iota for k in range(_L)]

        def do_transpose(b):
            src, dst = gbuf[b], sbuf[b]

            nq = d // _L

            @plsc.parallel_loop(0, (lanes // _L) * nq)
            def _(i):
                s0 = (i // nq) * _L
                c0 = (i % nq) * _L
                svec = s0 + dim_iota
                wsplat = jnp.full((_L,), c0 * lanes + s0, jnp.int32)
                for k in range(_L):
                    vals = plsc.load_gather(src, [svec, c0 + cperm[k]])
                    plsc.store_scatter(
                        dst, [zero_vec, zero_vec, wperm[k] + wsplat],
                        vals * scale)

        for b in range(_NBUF):
            start_gather(b, b)
        for b in range(_NBUF):
            wait_gather(b)
            do_transpose(b)
            start_scatter(b, b)
            start_gather(b + _NBUF, b)
        @pl.loop(_NBUF, t_len - _NBUF, step=_NBUF)
        def _(t0):
            for b in range(_NBUF):
                t = t0 + b
                wait_scatter(b)
                wait_gather(b)
                do_transpose(b)
                start_scatter(t, b)
                start_gather(t + _NBUF, b)
        for b in range(_NBUF):
            t = t_len - _NBUF + b
            wait_scatter(b)
            wait_gather(b)
            do_transpose(b)
            start_scatter(t, b)
        for b in range(_NBUF):
            wait_scatter(b)

    return emb


def kernel(x, table):
    n_seq, t_len = x.shape
    d = table.shape[1]
    out5 = _emb_kernel(n_seq, t_len, d)(x.T.astype(jnp.int32), table)
    return jnp.transpose(out5, (2, 4, 0, 1, 3)).reshape(n_seq, t_len, d)
